# Initial kernel scaffold; baseline (speedup 1.0000x reference)
import jax
import jax.numpy as jnp
from jax.experimental import pallas as pl


def kernel(x_o, x_a, edge_index, idx, W1, b1, a1, W2, b2, a2, mlp1_W, mlp1_b, disc_W, disc_b, dec1_W, dec1_b, dec2_W, dec2_b):
    raise NotImplementedError("write your pallas kernel here")


# speedup vs baseline: 3.6663x; 3.6663x over previous
import functools

import jax
import jax.numpy as jnp
from jax import lax
from jax.experimental import pallas as pl
from jax.experimental.pallas import tpu as pltpu
from jax.experimental.pallas import tpu_sc as plsc

N = 50000
NP = 50176
E = 800000
KCH = 128
NSUB = 16
NCHUNK = 392
EPAD = NSUB * NCHUNK * KCH
BN = 512
NB = NP // BN
STRIPE = NP // NSUB
B = 4096
OFFSET = 10367
CW = 8
NC1 = 128 // CW
NC2 = 64 // CW


@functools.cache
def _sc_kernels():
    mesh = plsc.VectorSubcoreMesh(core_axis_name="c", subcore_axis_name="s")
    cp = pltpu.CompilerParams(use_tc_tiling_on_sc=False)

    @functools.partial(
        pl.kernel,
        out_type=jax.ShapeDtypeStruct((NP,), jnp.float32),
        mesh=mesh,
        compiler_params=cp,
        scratch_types=[
            pltpu.VMEM((NCHUNK, KCH), jnp.int32),
            pltpu.VMEM((KCH,), jnp.float32),
            pltpu.VMEM((STRIPE,), jnp.float32),
            pltpu.VMEM_SHARED((NP,), jnp.float32),
            pltpu.SemaphoreType.DMA,
        ],
    )
    def _deg_sc(dst_hbm, deg_hbm, dst_v, ones_v, zbuf, acc, sem):
        s = lax.axis_index("s")
        c = lax.axis_index("c")
        pltpu.sync_copy(dst_hbm.at[s], dst_v)
        for i in range(KCH // 16):
            ones_v[pl.ds(i * 16, 16)] = jnp.ones((16,), jnp.float32)

        def zbody(i, _):
            zbuf[pl.ds(i * 16, 16)] = jnp.zeros((16,), jnp.float32)
            return 0

        lax.fori_loop(0, STRIPE // 16, zbody, 0)
        pltpu.sync_copy(zbuf, acc.at[pl.ds(s * STRIPE, STRIPE)])
        plsc.subcore_barrier()

        def body(j, _):
            pltpu.sync_copy(ones_v, acc.at[dst_v.at[j]], add=True)
            return 0

        lax.fori_loop(0, NCHUNK, body, 0)
        plsc.subcore_barrier()
        half = NP // (2 * NSUB)
        base = (c * NSUB + s) * half
        pltpu.sync_copy(acc.at[pl.ds(base, half)], zbuf.at[pl.ds(0, half)])
        pltpu.sync_copy(zbuf.at[pl.ds(0, half)], deg_hbm.at[pl.ds(base, half)])

    def _make_agg_sc(nfc):
        ZCH = STRIPE // 8

        @functools.partial(
            pl.kernel,
            out_type=jax.ShapeDtypeStruct((2 * nfc, NP, CW), jnp.float32),
            mesh=mesh,
            compiler_params=cp,
            scratch_types=[
                pltpu.VMEM((NCHUNK, KCH), jnp.int32),
                pltpu.VMEM((NCHUNK, KCH), jnp.int32),
                pltpu.VMEM((KCH, CW), jnp.float32),
                pltpu.VMEM((KCH, CW), jnp.float32),
                pltpu.VMEM((ZCH, CW), jnp.float32),
                pltpu.VMEM_SHARED((NP, CW), jnp.float32),
                pltpu.SemaphoreType.DMA,
                pltpu.SemaphoreType.DMA,
            ],
        )
        def _agg(y_hbm, src_hbm, dst_hbm, agg_hbm,
                 src_v, dst_v, buf0, buf1, zwbuf, acc, sem0, sem1):
            s = lax.axis_index("s")
            c = lax.axis_index("c")
            pltpu.sync_copy(src_hbm.at[s], src_v)
            pltpu.sync_copy(dst_hbm.at[s], dst_v)

            for fc in range(nfc):
                k = c * nfc + fc
                tab = y_hbm.at[k]
                out = agg_hbm.at[k]

                def z16(i, _):
                    zwbuf[pl.ds(2 * i, 2), :] = jnp.zeros((2, CW),
                                                          jnp.float32)
                    return 0

                lax.fori_loop(0, ZCH // 2, z16, 0)
                for z in range(8):
                    pltpu.sync_copy(
                        zwbuf, acc.at[pl.ds(s * STRIPE + z * ZCH, ZCH)])
                plsc.subcore_barrier()
                pltpu.async_copy(tab.at[src_v.at[0]], buf0, sem0)
                pltpu.async_copy(tab.at[src_v.at[1]], buf1, sem1)

                def body(i, _):
                    j = 2 * i
                    pltpu.make_async_copy(
                        tab.at[src_v.at[j]], buf0, sem0).wait()
                    pltpu.sync_copy(buf0, acc.at[dst_v.at[j]], add=True)
                    pltpu.async_copy(
                        tab.at[src_v.at[(j + 2) % NCHUNK]], buf0, sem0)
                    pltpu.make_async_copy(
                        tab.at[src_v.at[j + 1]], buf1, sem1).wait()
                    pltpu.sync_copy(buf1, acc.at[dst_v.at[j + 1]], add=True)
                    pltpu.async_copy(
                        tab.at[src_v.at[(j + 3) % NCHUNK]], buf1, sem1)
                    return 0

                lax.fori_loop(0, NCHUNK // 2, body, 0)
                pltpu.make_async_copy(tab.at[src_v.at[0]], buf0, sem0).wait()
                pltpu.make_async_copy(tab.at[src_v.at[1]], buf1, sem1).wait()
                plsc.subcore_barrier()
                for z in range(8):
                    base = s * STRIPE + z * ZCH
                    pltpu.sync_copy(acc.at[pl.ds(base, ZCH)], zwbuf)
                    pltpu.sync_copy(zwbuf, out.at[pl.ds(base, ZCH)])
                plsc.subcore_barrier()

        return _agg

    @functools.partial(
        pl.kernel,
        out_type=jax.ShapeDtypeStruct((2, B, 64), jnp.float32),
        mesh=mesh,
        compiler_params=cp,
        scratch_types=[
            pltpu.VMEM((KCH,), jnp.int32),
            pltpu.VMEM((KCH, 64), jnp.float32),
            pltpu.SemaphoreType.DMA,
        ],
    )
    def _dec_gather_sc(x2_hbm, idx_hbm, e_hbm, idx_v, ebuf, sem):
        s = lax.axis_index("s")
        c = lax.axis_index("c")
        w = s * 2 + c
        tab = x2_hbm.at[0]
        for e in range(2):
            pltpu.sync_copy(idx_hbm.at[e, w], idx_v)
            pltpu.async_copy(tab.at[idx_v], ebuf, sem).wait()
            pltpu.sync_copy(ebuf, e_hbm.at[e, pl.ds(w * KCH, KCH)])

    return _deg_sc, _make_agg_sc(NC1), _make_agg_sc(NC2), _dec_gather_sc


def _xw_body(x_ref, deg_ref, w_ref, y_ref):
    dinv = lax.rsqrt(deg_ref[...] + 1.0)
    xw = jnp.dot(x_ref[0], w_ref[...], preferred_element_type=jnp.float32)
    y = xw * dinv
    for i in range(NC1):
        y_ref[i] = y[:, CW * i:CW * (i + 1)]


_xw_call = pl.pallas_call(
    _xw_body,
    grid=(2, NB),
    in_specs=[
        pl.BlockSpec((1, BN, 128), lambda g, nb: (g, nb, 0)),
        pl.BlockSpec((BN, 1), lambda g, nb: (nb, 0)),
        pl.BlockSpec((128, 128), lambda g, nb: (0, 0)),
    ],
    out_specs=pl.BlockSpec((NC1, BN, CW), lambda g, nb: (g, nb, 0)),
    out_shape=jax.ShapeDtypeStruct((2 * NC1, NP, CW), jnp.float32),
)


def _l1_body(agg_ref, y_ref, deg_ref, w2_ref, b1_ref, a1_ref, y2_ref, acc_s):
    fc = pl.program_id(2)
    dinv = lax.rsqrt(deg_ref[...] + 1.0)
    b1 = b1_ref[pl.ds(fc, 1), :]
    a1 = a1_ref[pl.ds(fc, 1), :]
    pre = (agg_ref[0] + y_ref[0]) * dinv + b1
    h = jnp.where(pre >= 0.0, pre, a1 * pre)
    p = jnp.dot(h, w2_ref[...], preferred_element_type=jnp.float32)

    @pl.when(fc == 0)
    def _():
        acc_s[...] = p

    @pl.when(fc != 0)
    def _():
        acc_s[...] = acc_s[...] + p

    @pl.when(fc == NC1 - 1)
    def _():
        t = acc_s[...] * dinv
        for i in range(NC2):
            y2_ref[i] = t[:, CW * i:CW * (i + 1)]


_l1_call = pl.pallas_call(
    _l1_body,
    grid=(2, NB, NC1),
    in_specs=[
        pl.BlockSpec((1, BN, CW), lambda g, nb, fc: (g * NC1 + fc, nb, 0)),
        pl.BlockSpec((1, BN, CW), lambda g, nb, fc: (g * NC1 + fc, nb, 0)),
        pl.BlockSpec((BN, 1), lambda g, nb, fc: (nb, 0)),
        pl.BlockSpec((CW, 64), lambda g, nb, fc: (fc, 0)),
        pl.BlockSpec((NC1, CW), lambda g, nb, fc: (0, 0)),
        pl.BlockSpec((NC1, CW), lambda g, nb, fc: (0, 0)),
    ],
    out_specs=pl.BlockSpec((NC2, BN, CW), lambda g, nb, fc: (g, nb, 0)),
    out_shape=jax.ShapeDtypeStruct((2 * NC2, NP, CW), jnp.float32),
    scratch_shapes=[pltpu.VMEM((BN, 64), jnp.float32)],
)


def _l2_body(agg_ref, y_ref, deg_ref, b2_ref, a2_ref, x2_ref):
    dinv = lax.rsqrt(deg_ref[...] + 1.0)
    agg = jnp.concatenate([agg_ref[i] for i in range(NC2)], axis=1)
    y = jnp.concatenate([y_ref[i] for i in range(NC2)], axis=1)
    pre = (agg + y) * dinv + b2_ref[...]
    x2_ref[0] = jnp.where(pre >= 0.0, pre, a2_ref[...] * pre)


_l2_call = pl.pallas_call(
    _l2_body,
    grid=(2, NB),
    in_specs=[
        pl.BlockSpec((NC2, BN, CW), lambda g, nb: (g, nb, 0)),
        pl.BlockSpec((NC2, BN, CW), lambda g, nb: (g, nb, 0)),
        pl.BlockSpec((BN, 1), lambda g, nb: (nb, 0)),
        pl.BlockSpec((1, 64), lambda g, nb: (0, 0)),
        pl.BlockSpec((1, 64), lambda g, nb: (0, 0)),
    ],
    out_specs=pl.BlockSpec((1, BN, 64), lambda g, nb: (g, nb, 0)),
    out_shape=jax.ShapeDtypeStruct((2, NP, 64), jnp.float32),
)


def _vvec_body(x2_ref, mw_ref, mb_ref, dwt_ref, v_ref, ssum):
    nb = pl.program_id(0)
    rows = nb * BN + lax.broadcasted_iota(jnp.int32, (BN, 1), 0)
    xm = jnp.where(rows < N, x2_ref[0], 0.0)
    part = jnp.sum(xm, axis=0, keepdims=True)

    @pl.when(nb == 0)
    def _():
        ssum[...] = part

    @pl.when(nb != 0)
    def _():
        ssum[...] = ssum[...] + part

    @pl.when(nb == NB - 1)
    def _():
        c = jax.nn.sigmoid(ssum[...] / N)
        h_os = jnp.dot(c, mw_ref[...],
                       preferred_element_type=jnp.float32) + mb_ref[...]
        v_ref[...] = jnp.dot(h_os, dwt_ref[...],
                             preferred_element_type=jnp.float32)


_vvec_call = pl.pallas_call(
    _vvec_body,
    grid=(NB,),
    in_specs=[
        pl.BlockSpec((1, BN, 64), lambda nb: (0, nb, 0)),
        pl.BlockSpec((64, 64), lambda nb: (0, 0)),
        pl.BlockSpec((1, 64), lambda nb: (0, 0)),
        pl.BlockSpec((64, 64), lambda nb: (0, 0)),
    ],
    out_specs=pl.BlockSpec((1, 64), lambda nb: (0, 0)),
    out_shape=jax.ShapeDtypeStruct((1, 64), jnp.float32),
    scratch_shapes=[pltpu.VMEM((1, 64), jnp.float32)],
)


def _ret_body(x2o_ref, x2a_ref, v_ref, db_ref, sc1_ref, sc2_ref):
    v = v_ref[...]
    sc1_ref[...] = jnp.sum(x2o_ref[0] * v, axis=1, keepdims=True) + db_ref[...]
    sc2_ref[...] = jnp.sum(x2a_ref[0] * v, axis=1, keepdims=True) + db_ref[...]


_ret_call = pl.pallas_call(
    _ret_body,
    grid=(NB,),
    in_specs=[
        pl.BlockSpec((1, BN, 64), lambda nb: (0, nb, 0)),
        pl.BlockSpec((1, BN, 64), lambda nb: (1, nb, 0)),
        pl.BlockSpec((1, 64), lambda nb: (0, 0)),
        pl.BlockSpec((1, 1), lambda nb: (0, 0)),
    ],
    out_specs=[
        pl.BlockSpec((BN, 1), lambda nb: (nb, 0)),
        pl.BlockSpec((BN, 1), lambda nb: (nb, 0)),
    ],
    out_shape=[
        jax.ShapeDtypeStruct((NP, 1), jnp.float32),
        jax.ShapeDtypeStruct((NP, 1), jnp.float32),
    ],
)


def _dec_body(e1_ref, e2_ref, d1w_ref, d1b_ref, d2w_ref, d2b_ref,
              log1_ref, log_ref):
    e1 = e1_ref[0]
    e2 = e2_ref[0]
    d1w = d1w_ref[...]
    f1 = (jnp.dot(e1 + e2, d1w[:64], preferred_element_type=jnp.float32)
          + jnp.dot(e1 * e2, d1w[64:128], preferred_element_type=jnp.float32)
          + jnp.dot(e1, d1w[128:192], preferred_element_type=jnp.float32)
          + jnp.dot(e2, d1w[192:], preferred_element_type=jnp.float32)
          + d1b_ref[...])
    log1 = jnp.maximum(f1, 0.0)
    log1_ref[...] = log1
    log_ref[...] = jnp.sum(log1 * d2w_ref[...], axis=1,
                           keepdims=True) + d2b_ref[...]


_dec_call = pl.pallas_call(
    _dec_body,
    grid=(B // BN,),
    in_specs=[
        pl.BlockSpec((1, BN, 64), lambda nb: (0, nb, 0)),
        pl.BlockSpec((1, BN, 64), lambda nb: (1, nb, 0)),
        pl.BlockSpec((256, 64), lambda nb: (0, 0)),
        pl.BlockSpec((1, 64), lambda nb: (0, 0)),
        pl.BlockSpec((1, 64), lambda nb: (0, 0)),
        pl.BlockSpec((1, 1), lambda nb: (0, 0)),
    ],
    out_specs=[
        pl.BlockSpec((BN, 64), lambda nb: (nb, 0)),
        pl.BlockSpec((BN, 1), lambda nb: (nb, 0)),
    ],
    out_shape=[
        jax.ShapeDtypeStruct((B, 64), jnp.float32),
        jax.ShapeDtypeStruct((B, 1), jnp.float32),
    ],
)


def kernel(x_o, x_a, edge_index, idx, W1, b1, a1, W2, b2, a2,
           mlp1_W, mlp1_b, disc_W, disc_b, dec1_W, dec1_b, dec2_W, dec2_b):
    pad = EPAD - E
    ar = jnp.arange(pad, dtype=jnp.int32)
    src3 = jnp.concatenate([edge_index[0], ar % N]).reshape(NSUB, NCHUNK, KCH)
    dst3 = jnp.concatenate(
        [edge_index[1], N + (ar % KCH)]).reshape(NSUB, NCHUNK, KCH)

    _deg_sc, _agg_sc_l1, _agg_sc_l2, _dec_gather_sc = _sc_kernels()
    deg = _deg_sc(dst3)
    deg2 = deg.reshape(NP, 1)

    xcat = jnp.stack([
        jnp.pad(x_o, ((0, NP - N), (0, 0))),
        jnp.pad(x_a, ((0, NP - N), (0, 0))),
    ])
    y1 = _xw_call(xcat, deg2, W1)
    agg1 = _agg_sc_l1(y1, src3, dst3)
    y2 = _l1_call(agg1, y1, deg2, W2,
                  b1.reshape(NC1, CW), a1.reshape(NC1, CW))
    agg2 = _agg_sc_l2(y2, src3, dst3)
    x2 = _l2_call(agg2, y2, deg2,
                  b2.reshape(1, 64), a2.reshape(1, 64))

    v = _vvec_call(x2, mlp1_W, mlp1_b.reshape(1, 64), disc_W.T)
    sc1, sc2 = _ret_call(x2, x2, v, disc_b.reshape(1, 1))
    ret_os = jnp.concatenate([sc1[:N], sc2[:N]], axis=1)

    idxp = jnp.stack([idx[0], idx[1] + OFFSET]).reshape(2, 32, KCH)
    e12 = _dec_gather_sc(x2, idxp)
    log1, log = _dec_call(e12, e12, dec1_W, dec1_b.reshape(1, 64),
                          dec2_W.reshape(1, 64), dec2_b.reshape(1, 1))

    x2_o = x2[0, :N]
    return log, ret_os, x2_o, log1


# speedup vs baseline: 4.5314x; 1.2360x over previous
import functools

import jax
import jax.numpy as jnp
from jax import lax
from jax.experimental import pallas as pl
from jax.experimental.pallas import tpu as pltpu
from jax.experimental.pallas import tpu_sc as plsc

N = 50000
NP = 50176
E = 800000
KCH = 128
NSUB = 16
NCHUNK = 392
EPAD = NSUB * NCHUNK * KCH
BN = 512
NB = NP // BN
STRIPE = NP // NSUB
B = 4096
OFFSET = 10367
CW = 8
NC1 = 128 // CW
NC2 = 64 // CW


@functools.cache
def _sc_kernels():
    mesh = plsc.VectorSubcoreMesh(core_axis_name="c", subcore_axis_name="s")
    cp = pltpu.CompilerParams(use_tc_tiling_on_sc=False)

    @functools.partial(
        pl.kernel,
        out_type=jax.ShapeDtypeStruct((NP,), jnp.float32),
        mesh=mesh,
        compiler_params=cp,
        scratch_types=[
            pltpu.VMEM((NCHUNK, KCH), jnp.int32),
            pltpu.VMEM((KCH,), jnp.float32),
            pltpu.VMEM((STRIPE,), jnp.float32),
            pltpu.VMEM_SHARED((NP,), jnp.float32),
            pltpu.SemaphoreType.DMA,
        ],
    )
    def _deg_sc(dst_hbm, deg_hbm, dst_v, ones_v, zbuf, acc, sem):
        s = lax.axis_index("s")
        c = lax.axis_index("c")
        pltpu.sync_copy(dst_hbm.at[s], dst_v)
        for i in range(KCH // 16):
            ones_v[pl.ds(i * 16, 16)] = jnp.ones((16,), jnp.float32)

        def zbody(i, _):
            zbuf[pl.ds(i * 16, 16)] = jnp.zeros((16,), jnp.float32)
            return 0

        lax.fori_loop(0, STRIPE // 16, zbody, 0)
        pltpu.sync_copy(zbuf, acc.at[pl.ds(s * STRIPE, STRIPE)])
        plsc.subcore_barrier()

        def body(j, _):
            pltpu.sync_copy(ones_v, acc.at[dst_v.at[j]], add=True)
            return 0

        lax.fori_loop(0, NCHUNK, body, 0)
        plsc.subcore_barrier()
        half = NP // (2 * NSUB)
        base = (c * NSUB + s) * half
        pltpu.sync_copy(acc.at[pl.ds(base, half)], zbuf.at[pl.ds(0, half)])
        pltpu.sync_copy(zbuf.at[pl.ds(0, half)], deg_hbm.at[pl.ds(base, half)])

    def _make_agg_sc(nfc):
        ZCH = STRIPE // 8

        @functools.partial(
            pl.kernel,
            out_type=jax.ShapeDtypeStruct((2 * nfc, NP, CW), jnp.float32),
            mesh=mesh,
            compiler_params=cp,
            scratch_types=[
                pltpu.VMEM((NCHUNK, KCH), jnp.int32),
                pltpu.VMEM((NCHUNK, KCH), jnp.int32),
                pltpu.VMEM((KCH, CW), jnp.float32),
                pltpu.VMEM((KCH, CW), jnp.float32),
                pltpu.VMEM((ZCH, CW), jnp.float32),
                pltpu.VMEM_SHARED((NP, CW), jnp.float32),
                pltpu.SemaphoreType.DMA,
                pltpu.SemaphoreType.DMA,
            ],
        )
        def _agg(y_hbm, src_hbm, dst_hbm, agg_hbm,
                 src_v, dst_v, buf0, buf1, zwbuf, acc, sem0, sem1):
            s = lax.axis_index("s")
            c = lax.axis_index("c")
            pltpu.sync_copy(src_hbm.at[s], src_v)
            pltpu.sync_copy(dst_hbm.at[s], dst_v)

            for fc in range(nfc):
                k = c * nfc + fc
                tab = y_hbm.at[k]
                out = agg_hbm.at[k]

                def z16(i, _):
                    zwbuf[pl.ds(2 * i, 2), :] = jnp.zeros((2, CW),
                                                          jnp.float32)
                    return 0

                lax.fori_loop(0, ZCH // 2, z16, 0)
                for z in range(8):
                    pltpu.sync_copy(
                        zwbuf, acc.at[pl.ds(s * STRIPE + z * ZCH, ZCH)])
                plsc.subcore_barrier()
                pltpu.async_copy(tab.at[src_v.at[0]], buf0, sem0)
                pltpu.async_copy(tab.at[src_v.at[1]], buf1, sem1)

                def body(i, _):
                    j = 2 * i
                    pltpu.make_async_copy(
                        tab.at[src_v.at[j]], buf0, sem0).wait()
                    pltpu.sync_copy(buf0, acc.at[dst_v.at[j]], add=True)
                    pltpu.async_copy(
                        tab.at[src_v.at[(j + 2) % NCHUNK]], buf0, sem0)
                    pltpu.make_async_copy(
                        tab.at[src_v.at[j + 1]], buf1, sem1).wait()
                    pltpu.sync_copy(buf1, acc.at[dst_v.at[j + 1]], add=True)
                    pltpu.async_copy(
                        tab.at[src_v.at[(j + 3) % NCHUNK]], buf1, sem1)
                    return 0

                lax.fori_loop(0, NCHUNK // 2, body, 0)
                pltpu.make_async_copy(tab.at[src_v.at[0]], buf0, sem0).wait()
                pltpu.make_async_copy(tab.at[src_v.at[1]], buf1, sem1).wait()
                plsc.subcore_barrier()
                for z in range(8):
                    base = s * STRIPE + z * ZCH
                    pltpu.sync_copy(acc.at[pl.ds(base, ZCH)], zwbuf)
                    pltpu.sync_copy(zwbuf, out.at[pl.ds(base, ZCH)])
                plsc.subcore_barrier()

        return _agg

    @functools.partial(
        pl.kernel,
        out_type=jax.ShapeDtypeStruct((2, B, 64), jnp.float32),
        mesh=mesh,
        compiler_params=cp,
        scratch_types=[
            pltpu.VMEM((KCH,), jnp.int32),
            pltpu.VMEM((KCH, 64), jnp.float32),
            pltpu.SemaphoreType.DMA,
        ],
    )
    def _dec_gather_sc(x2_hbm, idx_hbm, e_hbm, idx_v, ebuf, sem):
        s = lax.axis_index("s")
        c = lax.axis_index("c")
        w = s * 2 + c
        tab = x2_hbm.at[0]
        for e in range(2):
            pltpu.sync_copy(idx_hbm.at[e, w], idx_v)
            pltpu.async_copy(tab.at[idx_v], ebuf, sem).wait()
            pltpu.sync_copy(ebuf, e_hbm.at[e, pl.ds(w * KCH, KCH)])

    return _deg_sc, _make_agg_sc(NC1), _make_agg_sc(NC2), _dec_gather_sc


def _xw_body(x_ref, deg_ref, w_ref, y_ref):
    dinv = lax.rsqrt(deg_ref[...] + 1.0)
    xw = jnp.dot(x_ref[0], w_ref[...], preferred_element_type=jnp.float32)
    y = xw * dinv
    for i in range(NC1):
        y_ref[i] = y[:, CW * i:CW * (i + 1)]


_xw_call = pl.pallas_call(
    _xw_body,
    grid=(2, NB),
    in_specs=[
        pl.BlockSpec((1, BN, 128), lambda g, nb: (g, nb, 0)),
        pl.BlockSpec((BN, 1), lambda g, nb: (nb, 0)),
        pl.BlockSpec((128, 128), lambda g, nb: (0, 0)),
    ],
    out_specs=pl.BlockSpec((NC1, BN, CW), lambda g, nb: (g, nb, 0)),
    out_shape=jax.ShapeDtypeStruct((2 * NC1, NP, CW), jnp.float32),
)


def _l1_body(*refs):
    agg_refs = refs[:NC1]
    y_refs = refs[NC1:2 * NC1]
    deg_ref, w2_ref, b1_ref, a1_ref = refs[2 * NC1:2 * NC1 + 4]
    y2_ref = refs[-1]
    dinv = lax.rsqrt(deg_ref[...] + 1.0)
    acc = jnp.zeros((BN, 64), jnp.float32)
    for i in range(NC1):
        b1 = b1_ref[pl.ds(i, 1), :]
        a1 = a1_ref[pl.ds(i, 1), :]
        pre = (agg_refs[i][0] + y_refs[i][0]) * dinv + b1
        h = jnp.where(pre >= 0.0, pre, a1 * pre)
        acc = acc + jnp.dot(h, w2_ref[pl.ds(CW * i, CW), :],
                            preferred_element_type=jnp.float32)
    t = acc * dinv
    for i in range(NC2):
        y2_ref[i] = t[:, CW * i:CW * (i + 1)]


def _chunk_spec(i):
    return pl.BlockSpec((1, BN, CW), lambda g, nb, i=i: (g * NC1 + i, nb, 0))


_l1_call = pl.pallas_call(
    _l1_body,
    grid=(2, NB),
    in_specs=(
        [_chunk_spec(i) for i in range(NC1)]
        + [_chunk_spec(i) for i in range(NC1)]
        + [
            pl.BlockSpec((BN, 1), lambda g, nb: (nb, 0)),
            pl.BlockSpec((128, 64), lambda g, nb: (0, 0)),
            pl.BlockSpec((NC1, CW), lambda g, nb: (0, 0)),
            pl.BlockSpec((NC1, CW), lambda g, nb: (0, 0)),
        ]
    ),
    out_specs=pl.BlockSpec((NC2, BN, CW), lambda g, nb: (g, nb, 0)),
    out_shape=jax.ShapeDtypeStruct((2 * NC2, NP, CW), jnp.float32),
)


def _l2_body(agg_ref, y_ref, deg_ref, b2_ref, a2_ref, x2_ref):
    dinv = lax.rsqrt(deg_ref[...] + 1.0)
    agg = jnp.concatenate([agg_ref[i] for i in range(NC2)], axis=1)
    y = jnp.concatenate([y_ref[i] for i in range(NC2)], axis=1)
    pre = (agg + y) * dinv + b2_ref[...]
    x2_ref[0] = jnp.where(pre >= 0.0, pre, a2_ref[...] * pre)


_l2_call = pl.pallas_call(
    _l2_body,
    grid=(2, NB),
    in_specs=[
        pl.BlockSpec((NC2, BN, CW), lambda g, nb: (g, nb, 0)),
        pl.BlockSpec((NC2, BN, CW), lambda g, nb: (g, nb, 0)),
        pl.BlockSpec((BN, 1), lambda g, nb: (nb, 0)),
        pl.BlockSpec((1, 64), lambda g, nb: (0, 0)),
        pl.BlockSpec((1, 64), lambda g, nb: (0, 0)),
    ],
    out_specs=pl.BlockSpec((1, BN, 64), lambda g, nb: (g, nb, 0)),
    out_shape=jax.ShapeDtypeStruct((2, NP, 64), jnp.float32),
)


def _vvec_body(x2_ref, mw_ref, mb_ref, dwt_ref, v_ref, ssum):
    nb = pl.program_id(0)
    rows = nb * BN + lax.broadcasted_iota(jnp.int32, (BN, 1), 0)
    xm = jnp.where(rows < N, x2_ref[0], 0.0)
    part = jnp.sum(xm, axis=0, keepdims=True)

    @pl.when(nb == 0)
    def _():
        ssum[...] = part

    @pl.when(nb != 0)
    def _():
        ssum[...] = ssum[...] + part

    @pl.when(nb == NB - 1)
    def _():
        c = jax.nn.sigmoid(ssum[...] / N)
        h_os = jnp.dot(c, mw_ref[...],
                       preferred_element_type=jnp.float32) + mb_ref[...]
        v_ref[...] = jnp.dot(h_os, dwt_ref[...],
                             preferred_element_type=jnp.float32)


_vvec_call = pl.pallas_call(
    _vvec_body,
    grid=(NB,),
    in_specs=[
        pl.BlockSpec((1, BN, 64), lambda nb: (0, nb, 0)),
        pl.BlockSpec((64, 64), lambda nb: (0, 0)),
        pl.BlockSpec((1, 64), lambda nb: (0, 0)),
        pl.BlockSpec((64, 64), lambda nb: (0, 0)),
    ],
    out_specs=pl.BlockSpec((1, 64), lambda nb: (0, 0)),
    out_shape=jax.ShapeDtypeStruct((1, 64), jnp.float32),
    scratch_shapes=[pltpu.VMEM((1, 64), jnp.float32)],
)


def _ret_body(x2o_ref, x2a_ref, v_ref, db_ref, sc1_ref, sc2_ref):
    v = v_ref[...]
    sc1_ref[...] = jnp.sum(x2o_ref[0] * v, axis=1, keepdims=True) + db_ref[...]
    sc2_ref[...] = jnp.sum(x2a_ref[0] * v, axis=1, keepdims=True) + db_ref[...]


_ret_call = pl.pallas_call(
    _ret_body,
    grid=(NB,),
    in_specs=[
        pl.BlockSpec((1, BN, 64), lambda nb: (0, nb, 0)),
        pl.BlockSpec((1, BN, 64), lambda nb: (1, nb, 0)),
        pl.BlockSpec((1, 64), lambda nb: (0, 0)),
        pl.BlockSpec((1, 1), lambda nb: (0, 0)),
    ],
    out_specs=[
        pl.BlockSpec((BN, 1), lambda nb: (nb, 0)),
        pl.BlockSpec((BN, 1), lambda nb: (nb, 0)),
    ],
    out_shape=[
        jax.ShapeDtypeStruct((NP, 1), jnp.float32),
        jax.ShapeDtypeStruct((NP, 1), jnp.float32),
    ],
)


def _dec_body(e1_ref, e2_ref, d1w_ref, d1b_ref, d2w_ref, d2b_ref,
              log1_ref, log_ref):
    e1 = e1_ref[0]
    e2 = e2_ref[0]
    d1w = d1w_ref[...]
    f1 = (jnp.dot(e1 + e2, d1w[:64], preferred_element_type=jnp.float32)
          + jnp.dot(e1 * e2, d1w[64:128], preferred_element_type=jnp.float32)
          + jnp.dot(e1, d1w[128:192], preferred_element_type=jnp.float32)
          + jnp.dot(e2, d1w[192:], preferred_element_type=jnp.float32)
          + d1b_ref[...])
    log1 = jnp.maximum(f1, 0.0)
    log1_ref[...] = log1
    log_ref[...] = jnp.sum(log1 * d2w_ref[...], axis=1,
                           keepdims=True) + d2b_ref[...]


_dec_call = pl.pallas_call(
    _dec_body,
    grid=(B // BN,),
    in_specs=[
        pl.BlockSpec((1, BN, 64), lambda nb: (0, nb, 0)),
        pl.BlockSpec((1, BN, 64), lambda nb: (1, nb, 0)),
        pl.BlockSpec((256, 64), lambda nb: (0, 0)),
        pl.BlockSpec((1, 64), lambda nb: (0, 0)),
        pl.BlockSpec((1, 64), lambda nb: (0, 0)),
        pl.BlockSpec((1, 1), lambda nb: (0, 0)),
    ],
    out_specs=[
        pl.BlockSpec((BN, 64), lambda nb: (nb, 0)),
        pl.BlockSpec((BN, 1), lambda nb: (nb, 0)),
    ],
    out_shape=[
        jax.ShapeDtypeStruct((B, 64), jnp.float32),
        jax.ShapeDtypeStruct((B, 1), jnp.float32),
    ],
)


def kernel(x_o, x_a, edge_index, idx, W1, b1, a1, W2, b2, a2,
           mlp1_W, mlp1_b, disc_W, disc_b, dec1_W, dec1_b, dec2_W, dec2_b):
    pad = EPAD - E
    ar = jnp.arange(pad, dtype=jnp.int32)
    src3 = jnp.concatenate([edge_index[0], ar % N]).reshape(NSUB, NCHUNK, KCH)
    dst3 = jnp.concatenate(
        [edge_index[1], N + (ar % KCH)]).reshape(NSUB, NCHUNK, KCH)

    _deg_sc, _agg_sc_l1, _agg_sc_l2, _dec_gather_sc = _sc_kernels()
    deg = _deg_sc(dst3)
    deg2 = deg.reshape(NP, 1)

    xcat = jnp.stack([
        jnp.pad(x_o, ((0, NP - N), (0, 0))),
        jnp.pad(x_a, ((0, NP - N), (0, 0))),
    ])
    y1 = _xw_call(xcat, deg2, W1)
    agg1 = _agg_sc_l1(y1, src3, dst3)
    y2 = _l1_call(*([agg1] * NC1), *([y1] * NC1), deg2, W2,
                  b1.reshape(NC1, CW), a1.reshape(NC1, CW))
    agg2 = _agg_sc_l2(y2, src3, dst3)
    x2 = _l2_call(agg2, y2, deg2,
                  b2.reshape(1, 64), a2.reshape(1, 64))

    v = _vvec_call(x2, mlp1_W, mlp1_b.reshape(1, 64), disc_W.T)
    sc1, sc2 = _ret_call(x2, x2, v, disc_b.reshape(1, 1))
    ret_os = jnp.concatenate([sc1[:N], sc2[:N]], axis=1)

    idxp = jnp.stack([idx[0], idx[1] + OFFSET]).reshape(2, 32, KCH)
    e12 = _dec_gather_sc(x2, idxp)
    log1, log = _dec_call(e12, e12, dec1_W, dec1_b.reshape(1, 64),
                          dec2_W.reshape(1, 64), dec2_b.reshape(1, 1))

    x2_o = x2[0, :N]
    return log, ret_os, x2_o, log1
